# baseline (speedup 1.0000x reference)
import functools

import jax
import jax.numpy as jnp
from jax.experimental import pallas as pl
from jax.experimental.pallas import tpu as pltpu

_INTERPRET = False


def _moe_body(x_hbm, rw_ref, w1_hbm, w2_hbm, out_hbm, loss_ref,
              x_scr, w1_scr, w2_scr, out_scr, xsem, wsem, osem,
              *, n_exp, width, n_tok, gt, nck):
    d = x_scr.shape[1]
    d_ff = w1_scr.shape[1]
    ntile = n_tok // gt
    cw1 = d_ff // nck
    cw2 = d // nck

    def xcopy(j):
        return pltpu.make_async_copy(x_hbm.at[pl.ds(j * gt, gt), :],
                                     x_scr.at[pl.ds(j * gt, gt), :], xsem.at[j])

    def w1copy(c):
        return pltpu.make_async_copy(w1_hbm.at[:, pl.ds(c * cw1, cw1)],
                                     w1_scr.at[:, pl.ds(c * cw1, cw1)], wsem.at[c])

    def w2copy(c):
        return pltpu.make_async_copy(w2_hbm.at[pl.ds(c * cw1, cw1), :],
                                     w2_scr.at[pl.ds(c * cw1, cw1), :], wsem.at[nck + c])

    def ocopy(j):
        return pltpu.make_async_copy(out_scr.at[pl.ds(j * gt, gt), :],
                                     out_hbm.at[pl.ds(j * gt, gt), :], osem.at[j])

    xcopy(0).start()
    for c in range(nck):
        w1copy(c).start()
        w2copy(c).start()
        if 1 + c < ntile:
            xcopy(1 + c).start()
    for j in range(1 + nck, ntile):
        xcopy(j).start()

    p_acc = jnp.zeros((1, n_exp), jnp.float32)
    c_acc = jnp.zeros((1, n_exp), jnp.float32)
    z_acc = jnp.float32(0.0)

    for j in range(ntile):
        xcopy(j).wait()
        xj = x_scr[j * gt:(j + 1) * gt, :]

        logits = jax.lax.dot_general(xj, rw_ref[...], (((1,), (1,)), ((), ())),
                                     preferred_element_type=jnp.float32)
        probs = jax.nn.sigmoid(logits)

        lanes = jax.lax.broadcasted_iota(jnp.int32, probs.shape, 1)
        enc = jax.lax.bitcast_convert_type(probs, jnp.int32)
        enc = (enc & ~7) | (n_exp - 1 - lanes)
        e1 = jnp.max(enc, axis=1, keepdims=True)
        i1 = (n_exp - 1) - (e1 & 7)
        enc2 = jnp.where(lanes == i1, jnp.int32(0), enc)
        e2 = jnp.max(enc2, axis=1, keepdims=True)
        i2 = (n_exp - 1) - (e2 & 7)
        v1 = jax.lax.bitcast_convert_type(e1, jnp.float32)
        v2 = jax.lax.bitcast_convert_type(e2, jnp.float32)
        ssum = v1 + v2 + 1e-20
        oh1 = (lanes == i1).astype(jnp.float32)
        oh2 = (lanes == i2).astype(jnp.float32)
        combine = (oh1 * v1 + oh2 * v2) / ssum

        m = jnp.max(logits, axis=1, keepdims=True)
        lse = m + jnp.log(jnp.sum(jnp.exp(logits - m), axis=1, keepdims=True))
        z_acc = z_acc + jnp.sum(lse * lse)
        p_acc = p_acc + jnp.sum(probs, axis=0, keepdims=True)
        c_acc = c_acc + jnp.sum(oh1 + oh2, axis=0, keepdims=True)

        yj = jnp.zeros((gt, d), jnp.float32)
        for c in range(nck):
            if j == 0:
                w1copy(c).wait()
                w2copy(c).wait()
            hc = jnp.dot(xj, w1_scr[:, c * cw1:(c + 1) * cw1],
                         preferred_element_type=jnp.float32)
            lane_e = (jax.lax.broadcasted_iota(jnp.int32, hc.shape, 1)
                      + c * cw1) // width
            c_exp = jnp.zeros_like(hc)
            for e in range(c * cw1 // width, (c + 1) * cw1 // width):
                c_exp = jnp.where(lane_e == e, combine[:, e:e + 1], c_exp)
            ac = jnp.square(jnp.maximum(hc, 0.0)) * c_exp
            yj = yj + jnp.dot(ac, w2_scr[c * cw1:(c + 1) * cw1, :],
                              preferred_element_type=jnp.float32)
        out_scr[j * gt:(j + 1) * gt, :] = yj
        ocopy(j).start()

    for j in range(ntile):
        ocopy(j).wait()

    p_i = p_acc / n_tok
    f_i = c_acc / (2.0 * n_tok)
    z = z_acc / n_tok
    lb = n_exp * jnp.sum(f_i * p_i)
    closs = jnp.sum(p_acc) / n_tok
    loss_ref[0:1, :] = f_i
    loss_ref[1:2, :] = jnp.full((1, n_exp), z, jnp.float32)
    loss_ref[2:3, :] = jnp.full((1, n_exp), lb, jnp.float32)
    loss_ref[3:4, :] = jnp.full((1, n_exp), closs, jnp.float32)
    loss_ref[4:8, :] = jnp.zeros((4, n_exp), jnp.float32)


def kernel(x, router_w, w1, w2):
    b, s, d = x.shape
    n_exp, _ = router_w.shape
    total_w = w1.shape[1]
    width = total_w // n_exp
    t = b * s
    gt = 256
    nck = 4

    x_flat = x.reshape(t, d)
    body = functools.partial(_moe_body, n_exp=n_exp, width=width, n_tok=t,
                             gt=gt, nck=nck)
    hbm = pltpu.MemorySpace.HBM
    out_flat, lossbuf = pl.pallas_call(
        body,
        in_specs=[
            pl.BlockSpec(memory_space=hbm),
            pl.BlockSpec((n_exp, d), lambda: (0, 0)),
            pl.BlockSpec(memory_space=hbm),
            pl.BlockSpec(memory_space=hbm),
        ],
        out_specs=[
            pl.BlockSpec(memory_space=hbm),
            pl.BlockSpec((8, n_exp), lambda: (0, 0)),
        ],
        out_shape=[
            jax.ShapeDtypeStruct((t, d), jnp.float32),
            jax.ShapeDtypeStruct((8, n_exp), jnp.float32),
        ],
        scratch_shapes=[
            pltpu.VMEM((t, d), jnp.float32),
            pltpu.VMEM((d, total_w), jnp.float32),
            pltpu.VMEM((total_w, d), jnp.float32),
            pltpu.VMEM((t, d), jnp.float32),
            pltpu.SemaphoreType.DMA((t // gt,)),
            pltpu.SemaphoreType.DMA((2 * nck,)),
            pltpu.SemaphoreType.DMA((t // gt,)),
        ],
        interpret=_INTERPRET,
    )(x_flat, router_w, w1, w2)

    output = out_flat.reshape(b, s, d)
    f_i = lossbuf[0]
    z = lossbuf[1, 0]
    lb = lossbuf[2, 0]
    cl = lossbuf[3, 0]
    return (output, z, lb, cl, f_i)


# speedup vs baseline: 1.1424x; 1.1424x over previous
import functools

import jax
import jax.numpy as jnp
from jax.experimental import pallas as pl
from jax.experimental.pallas import tpu as pltpu

_INTERPRET = False


def _moe_body(x_ref, rw_ref, w1_ref, w2_ref, out_ref, loss_ref, acc_ref,
              *, n_exp, width, n_tok):
    j = pl.program_id(0)
    nt = pl.num_programs(0)
    x = x_ref[...]

    logits = jax.lax.dot_general(x, rw_ref[...], (((1,), (1,)), ((), ())),
                                 preferred_element_type=jnp.float32)

    h = jnp.dot(x, w1_ref[...], preferred_element_type=jnp.float32)

    probs = jax.nn.sigmoid(logits)

    lanes = jax.lax.broadcasted_iota(jnp.int32, probs.shape, 1)
    enc = jax.lax.bitcast_convert_type(probs, jnp.int32)
    enc = (enc & ~7) | (n_exp - 1 - lanes)
    e1 = jnp.max(enc, axis=1, keepdims=True)
    i1 = (n_exp - 1) - (e1 & 7)
    enc2 = jnp.where(lanes == i1, jnp.int32(0), enc)
    e2 = jnp.max(enc2, axis=1, keepdims=True)
    i2 = (n_exp - 1) - (e2 & 7)
    v1 = jax.lax.bitcast_convert_type(e1, jnp.float32)
    v2 = jax.lax.bitcast_convert_type(e2, jnp.float32)
    ssum = v1 + v2 + 1e-20
    oh1 = (lanes == i1).astype(jnp.float32)
    oh2 = (lanes == i2).astype(jnp.float32)
    combine = (oh1 * v1 + oh2 * v2) / ssum

    lane_e = jax.lax.broadcasted_iota(jnp.int32, h.shape, 1) // width
    c_exp = jnp.zeros_like(h)
    for e in range(n_exp):
        c_exp = jnp.where(lane_e == e, combine[:, e:e + 1], c_exp)

    a = jnp.square(jnp.maximum(h, 0.0)) * c_exp
    out_ref[...] = jnp.dot(a, w2_ref[...], preferred_element_type=jnp.float32)

    m = jnp.max(logits, axis=1, keepdims=True)
    lse = m + jnp.log(jnp.sum(jnp.exp(logits - m), axis=1, keepdims=True))
    z_part = jnp.sum(lse * lse)

    @pl.when(j == 0)
    def _():
        acc_ref[...] = jnp.zeros_like(acc_ref)

    acc_ref[0:1, :] += jnp.sum(probs, axis=0, keepdims=True)
    acc_ref[1:2, :] += jnp.sum(oh1 + oh2, axis=0, keepdims=True)
    acc_ref[2:3, :] += jnp.full((1, n_exp), z_part, jnp.float32)

    @pl.when(j == nt - 1)
    def _():
        p_i = acc_ref[0:1, :] / n_tok
        f_i = acc_ref[1:2, :] / (2.0 * n_tok)
        z = acc_ref[2, 0] / n_tok
        lb = n_exp * jnp.sum(f_i * p_i)
        closs = jnp.sum(acc_ref[0:1, :]) / n_tok
        loss_ref[0:1, :] = f_i
        loss_ref[1:2, :] = jnp.full((1, n_exp), z, jnp.float32)
        loss_ref[2:3, :] = jnp.full((1, n_exp), lb, jnp.float32)
        loss_ref[3:4, :] = jnp.full((1, n_exp), closs, jnp.float32)
        loss_ref[4:8, :] = jnp.zeros((4, n_exp), jnp.float32)


def kernel(x, router_w, w1, w2):
    b, s, d = x.shape
    n_exp, _ = router_w.shape
    total_w = w1.shape[1]
    width = total_w // n_exp
    t = b * s
    gt = 256
    nt = t // gt

    x_flat = x.reshape(t, d)
    body = functools.partial(_moe_body, n_exp=n_exp, width=width, n_tok=t)
    out_flat, lossbuf = pl.pallas_call(
        body,
        grid=(nt,),
        in_specs=[
            pl.BlockSpec((gt, d), lambda j: (j, 0)),
            pl.BlockSpec((n_exp, d), lambda j: (0, 0)),
            pl.BlockSpec((d, total_w), lambda j: (0, 0)),
            pl.BlockSpec((total_w, d), lambda j: (0, 0)),
        ],
        out_specs=[
            pl.BlockSpec((gt, d), lambda j: (j, 0)),
            pl.BlockSpec((8, n_exp), lambda j: (0, 0)),
        ],
        out_shape=[
            jax.ShapeDtypeStruct((t, d), jnp.float32),
            jax.ShapeDtypeStruct((8, n_exp), jnp.float32),
        ],
        scratch_shapes=[pltpu.VMEM((8, n_exp), jnp.float32)],
        interpret=_INTERPRET,
    )(x_flat, router_w, w1, w2)

    output = out_flat.reshape(b, s, d)
    f_i = lossbuf[0]
    z = lossbuf[1, 0]
    lb = lossbuf[2, 0]
    cl = lossbuf[3, 0]
    return (output, z, lb, cl, f_i)


# speedup vs baseline: 1.1694x; 1.0237x over previous
import functools

import jax
import jax.numpy as jnp
from jax.experimental import pallas as pl
from jax.experimental.pallas import tpu as pltpu

_INTERPRET = False


def _moe_body(x_ref, rw_ref, w1_ref, w2_ref, out_ref, loss_ref, acc_ref,
              *, n_exp, width, n_tok):
    j = pl.program_id(0)
    nt = pl.num_programs(0)
    x = x_ref[...]

    logits = jax.lax.dot_general(x, rw_ref[...], (((1,), (1,)), ((), ())),
                                 preferred_element_type=jnp.float32)

    h = jnp.dot(x, w1_ref[...], preferred_element_type=jnp.float32)

    probs = jax.nn.sigmoid(logits)

    lanes = jax.lax.broadcasted_iota(jnp.int32, probs.shape, 1)
    enc = jax.lax.bitcast_convert_type(probs, jnp.int32)
    enc = (enc & ~7) | (n_exp - 1 - lanes)
    e1 = jnp.max(enc, axis=1, keepdims=True)
    i1 = (n_exp - 1) - (e1 & 7)
    enc2 = jnp.where(lanes == i1, jnp.int32(0), enc)
    e2 = jnp.max(enc2, axis=1, keepdims=True)
    i2 = (n_exp - 1) - (e2 & 7)
    v1 = jax.lax.bitcast_convert_type(e1, jnp.float32)
    v2 = jax.lax.bitcast_convert_type(e2, jnp.float32)
    ssum = v1 + v2 + 1e-20
    oh1 = (lanes == i1).astype(jnp.float32)
    oh2 = (lanes == i2).astype(jnp.float32)
    combine = (oh1 * v1 + oh2 * v2) / ssum

    lane_e = jax.lax.broadcasted_iota(jnp.int32, h.shape, 1) // width
    c_exp = jnp.zeros_like(h)
    for e in range(n_exp):
        c_exp = jnp.where(lane_e == e, combine[:, e:e + 1], c_exp)

    a = jnp.square(jnp.maximum(h, 0.0)) * c_exp
    out_ref[...] = jnp.dot(a, w2_ref[...], preferred_element_type=jnp.float32)

    m = jnp.max(logits, axis=1, keepdims=True)
    lse = m + jnp.log(jnp.sum(jnp.exp(logits - m), axis=1, keepdims=True))
    z_part = jnp.sum(lse * lse)

    @pl.when(j == 0)
    def _():
        acc_ref[...] = jnp.zeros_like(acc_ref)

    acc_ref[0:1, :] += jnp.sum(probs, axis=0, keepdims=True)
    acc_ref[1:2, :] += jnp.sum(oh1 + oh2, axis=0, keepdims=True)
    acc_ref[2:3, :] += jnp.full((1, n_exp), z_part, jnp.float32)

    @pl.when(j == nt - 1)
    def _():
        p_i = acc_ref[0:1, :] / n_tok
        f_i = acc_ref[1:2, :] / (2.0 * n_tok)
        z = acc_ref[2, 0] / n_tok
        lb = n_exp * jnp.sum(f_i * p_i)
        closs = jnp.sum(acc_ref[0:1, :]) / n_tok
        loss_ref[0:1, :] = f_i
        loss_ref[1:2, :] = jnp.full((1, n_exp), z, jnp.float32)
        loss_ref[2:3, :] = jnp.full((1, n_exp), lb, jnp.float32)
        loss_ref[3:4, :] = jnp.full((1, n_exp), closs, jnp.float32)
        loss_ref[4:8, :] = jnp.zeros((4, n_exp), jnp.float32)


def kernel(x, router_w, w1, w2):
    b, s, d = x.shape
    n_exp, _ = router_w.shape
    total_w = w1.shape[1]
    width = total_w // n_exp
    t = b * s
    gt = 512
    nt = t // gt

    x_flat = x.reshape(t, d)
    body = functools.partial(_moe_body, n_exp=n_exp, width=width, n_tok=t)
    out_flat, lossbuf = pl.pallas_call(
        body,
        grid=(nt,),
        in_specs=[
            pl.BlockSpec((gt, d), lambda j: (j, 0)),
            pl.BlockSpec((n_exp, d), lambda j: (0, 0)),
            pl.BlockSpec((d, total_w), lambda j: (0, 0)),
            pl.BlockSpec((total_w, d), lambda j: (0, 0)),
        ],
        out_specs=[
            pl.BlockSpec((gt, d), lambda j: (j, 0)),
            pl.BlockSpec((8, n_exp), lambda j: (0, 0)),
        ],
        out_shape=[
            jax.ShapeDtypeStruct((t, d), jnp.float32),
            jax.ShapeDtypeStruct((8, n_exp), jnp.float32),
        ],
        scratch_shapes=[pltpu.VMEM((8, n_exp), jnp.float32)],
        interpret=_INTERPRET,
    )(x_flat, router_w, w1, w2)

    output = out_flat.reshape(b, s, d)
    f_i = lossbuf[0]
    z = lossbuf[1, 0]
    lb = lossbuf[2, 0]
    cl = lossbuf[3, 0]
    return (output, z, lb, cl, f_i)
